# baseline (speedup 1.0000x reference)
import functools

import jax
import jax.numpy as jnp
from jax import lax
from jax.experimental import pallas as pl
from jax.experimental.pallas import tpu as pltpu
from jax.experimental.pallas import tpu_sc as plsc

_F = 26
_CT = 125
_CCH = 25
_NCH = _CT // _CCH
_NSLOT = 5


def _sc_body(x_hbm, z_hbm, out_hbm, xall, posstore, buf, sems):
    wid = lax.axis_index("c") * 16 + lax.axis_index("s")
    lanes = lax.broadcasted_iota(jnp.int32, (16,), 0)
    ones16 = jnp.full((16,), 1.0, jnp.float32)
    zeros16 = jnp.zeros((16,), jnp.float32)

    for slot in range(_NSLOT):
        pltpu.sync_copy(z_hbm, buf.at[pl.ds(slot * _CCH, _CCH)])

    def _scatter(p, val):
        plsc.store_scatter(buf, [p >> 10, (p >> 7) & 7, p & 127], val)

    def _chunk(k, fprev):
        f = k // _NCH
        j = k - f * _NCH
        slot = lax.rem(k, _NSLOT)
        base = slot * _CCH * 1024
        cc0 = j * _CCH

        @pl.when(k >= _NSLOT)
        def _retire():
            pltpu.make_async_copy(
                buf.at[pl.ds(0, _CCH)],
                out_hbm.at[0, pl.ds(0, _CCH), 0],
                sems.at[slot],
            ).wait()
            for v in range(8):
                p = posstore[slot, pl.ds(v * 16, 16)]
                _scatter(p, zeros16)

        @pl.when(f != fprev)
        def _load_row():
            pltpu.sync_copy(x_hbm.at[f, wid], xall)

        for v in range(8):
            idx = xall[pl.ds(v * 16, 16)]
            cc = idx >> 3
            hit = (cc >= cc0) & (cc < cc0 + _CCH)
            p = base + ((cc - cc0) << 10) + ((idx & 7) << 7) + v * 16 + lanes
            p = jnp.where(hit, p, 125 * 1024 + lanes)
            posstore[slot, pl.ds(v * 16, 16)] = p
            _scatter(p, ones16)

        pltpu.make_async_copy(
            buf.at[pl.ds(slot * _CCH, _CCH)],
            out_hbm.at[f, pl.ds(cc0, _CCH), wid],
            sems.at[slot],
        ).start()
        return f

    lax.fori_loop(0, _F * _NCH, _chunk, jnp.int32(-1))

    for slot in range(_NSLOT):
        pltpu.make_async_copy(
            buf.at[pl.ds(0, _CCH)],
            out_hbm.at[0, pl.ds(0, _CCH), 0],
            sems.at[slot],
        ).wait()


def kernel(x):
    x = x.astype(jnp.int32)
    batch, feats = x.shape
    x_t3 = x.T.reshape(feats, 32, 128)
    zeros = jnp.zeros((_CCH, 8, 128), jnp.float32)
    mesh = plsc.VectorSubcoreMesh(core_axis_name="c", subcore_axis_name="s")
    run = functools.partial(
        pl.kernel,
        mesh=mesh,
        out_type=jax.ShapeDtypeStruct((_F, _CT, 32, 8, 128), jnp.float32),
        compiler_params=pltpu.CompilerParams(needs_layout_passes=False),
        scratch_types=[
            pltpu.VMEM((128,), jnp.int32),
            pltpu.VMEM((_NSLOT, 128), jnp.int32),
            pltpu.VMEM((_NSLOT * _CCH + 1, 8, 128), jnp.float32),
            pltpu.SemaphoreType.DMA((_NSLOT,)),
        ],
    )(_sc_body)
    a = run(x_t3, zeros)
    return a.transpose(2, 4, 0, 1, 3).reshape(batch, feats, 1000)


# speedup vs baseline: 1.0351x; 1.0351x over previous
import functools

import jax
import jax.numpy as jnp
from jax import lax
from jax.experimental import pallas as pl
from jax.experimental.pallas import tpu as pltpu
from jax.experimental.pallas import tpu_sc as plsc

_F = 26
_CT = 125
_CCH = 25
_NCH = _CT // _CCH
_NSLOT = 4


def _sc_body(x_hbm, z_hbm, out_hbm, xall, posstore, buf, sems):
    wid = lax.axis_index("c") * 16 + lax.axis_index("s")
    lanes = lax.broadcasted_iota(jnp.int32, (16,), 0)
    ones16 = jnp.full((16,), 1.0, jnp.float32)
    zeros16 = jnp.zeros((16,), jnp.float32)

    for slot in range(_NSLOT):
        pltpu.sync_copy(z_hbm, buf.at[slot, pl.ds(0, _CCH)])
    pltpu.sync_copy(x_hbm.at[pl.ds(0, _F), wid], xall)

    def _scatter(slotv, p, val):
        plsc.store_scatter(buf, [slotv, p >> 10, (p >> 7) & 7, p & 127], val)

    def _chunk(k, c):
        f = k // _NCH
        j = k - f * _NCH
        slot = k & (_NSLOT - 1)
        slotv = jnp.full((16,), slot, jnp.int32)
        cc0 = j * _CCH

        @pl.when(k >= _NSLOT)
        def _retire():
            pltpu.make_async_copy(
                buf.at[slot, pl.ds(0, _CCH)],
                out_hbm.at[0, pl.ds(0, _CCH), 0],
                sems.at[slot],
            ).wait()
            for v in range(8):
                p = posstore[slot, pl.ds(v * 16, 16)]
                _scatter(slotv, p, zeros16)

        for v in range(8):
            idx = xall[f, pl.ds(v * 16, 16)]
            cc = idx >> 3
            hit = (cc >= cc0) & (cc < cc0 + _CCH)
            p = ((cc - cc0) << 10) + ((idx & 7) << 7) + v * 16 + lanes
            p = jnp.where(hit, p, _CCH * 1024 + lanes)
            posstore[slot, pl.ds(v * 16, 16)] = p
            _scatter(slotv, p, ones16)

        pltpu.make_async_copy(
            buf.at[slot, pl.ds(0, _CCH)],
            out_hbm.at[f, pl.ds(cc0, _CCH), wid],
            sems.at[slot],
        ).start()
        return c

    lax.fori_loop(0, _F * _NCH, _chunk, 0)

    for slot in range(_NSLOT):
        pltpu.make_async_copy(
            buf.at[slot, pl.ds(0, _CCH)],
            out_hbm.at[0, pl.ds(0, _CCH), 0],
            sems.at[slot],
        ).wait()


def kernel(x):
    x = x.astype(jnp.int32)
    batch, feats = x.shape
    x_t3 = x.T.reshape(feats, 32, 128)
    zeros = jnp.zeros((_CCH, 8, 128), jnp.float32)
    mesh = plsc.VectorSubcoreMesh(core_axis_name="c", subcore_axis_name="s")
    run = functools.partial(
        pl.kernel,
        mesh=mesh,
        out_type=jax.ShapeDtypeStruct((_F, _CT, 32, 8, 128), jnp.float32),
        compiler_params=pltpu.CompilerParams(needs_layout_passes=False),
        scratch_types=[
            pltpu.VMEM((_F, 128), jnp.int32),
            pltpu.VMEM((_NSLOT, 128), jnp.int32),
            pltpu.VMEM((_NSLOT, _CCH + 1, 8, 128), jnp.float32),
            pltpu.SemaphoreType.DMA((_NSLOT,)),
        ],
    )(_sc_body)
    a = run(x_t3, zeros)
    return a.transpose(2, 4, 0, 1, 3).reshape(batch, feats, 1000)
